# Initial kernel scaffold; baseline (speedup 1.0000x reference)
import jax
import jax.numpy as jnp
from jax.experimental import pallas as pl


def kernel(feat_user, feat_item, W_follows, b_follows, W_clicks, b_clicks, W_bought, b_bought, edge_index_follows, edge_index_clicks, edge_index_bought):
    raise NotImplementedError("write your pallas kernel here")


# speedup vs baseline: 6.4432x; 6.4432x over previous
import functools

import jax
import jax.numpy as jnp
from jax import lax
from jax.experimental import pallas as pl
from jax.experimental.pallas import tpu as pltpu
from jax.experimental.pallas import tpu_sc as plsc

N_USER = 10000
N_ITEM = 10000
E = 320000
D = 128
CNT_W = 16

NC, NS = 2, 16
NW = NC * NS
E_PER_W = E // NW
CHUNK = 80
NCHUNK = E_PER_W // CHUNK
RPT = 624
TAIL = N_USER - NS * RPT


def _sc_body(feat_user, feat_item, z_s, z_c, src_f, dst_f, src_c, dst_c,
             src_b, dst_b,
             out_sf, out_cf, out_sc, out_cc, out_sb, out_cb,
             acc_s, acc_c, sidx, didx, rows, ones, sem):
    c = lax.axis_index("c")
    s = lax.axis_index("s")
    wid = c * NS + s
    row0 = pl.multiple_of(s * RPT, 8)
    tail0 = NS * RPT

    lane = lax.iota(jnp.int32, 16)
    one_vec = jnp.where(lane == 0, 1.0, 0.0).astype(jnp.float32)
    zvec = jnp.zeros((16,), jnp.float32)

    def fill_ones(i, carry):
        ones[i, :] = one_vec
        return carry

    lax.fori_loop(0, CHUNK, fill_ones, 0)

    def run_relation(feat, src_r, dst_r, out_s, out_c):
        pltpu.sync_copy(z_s, acc_s.at[pl.ds(row0, RPT)])
        pltpu.sync_copy(z_c, acc_c.at[pl.ds(row0, RPT)])

        @pl.when(s == 0)
        def _zero_tail():
            pltpu.sync_copy(z_s.at[pl.ds(0, TAIL)], acc_s.at[pl.ds(tail0, TAIL)])
            pltpu.sync_copy(z_c.at[pl.ds(0, TAIL)], acc_c.at[pl.ds(tail0, TAIL)])

        pltpu.sync_copy(src_r.at[wid], sidx)
        pltpu.sync_copy(dst_r.at[wid], didx)
        plsc.subcore_barrier()

        def chunk(j, carry):
            pltpu.async_copy(feat.at[sidx.at[j]], rows, sem).wait()
            pltpu.sync_copy(rows, acc_s.at[didx.at[j]], add=True)
            pltpu.sync_copy(ones, acc_c.at[didx.at[j]], add=True)
            return carry

        lax.fori_loop(0, NCHUNK, chunk, 0)
        plsc.subcore_barrier()
        pltpu.sync_copy(acc_s.at[pl.ds(row0, RPT)],
                        out_s.at[c, pl.ds(row0, RPT)])
        pltpu.sync_copy(acc_c.at[pl.ds(row0, RPT)],
                        out_c.at[c, pl.ds(row0, RPT)])

        @pl.when(s == 0)
        def _out_tail():
            pltpu.sync_copy(acc_s.at[pl.ds(tail0, TAIL)],
                            out_s.at[c, pl.ds(tail0, TAIL)])
            pltpu.sync_copy(acc_c.at[pl.ds(tail0, TAIL)],
                            out_c.at[c, pl.ds(tail0, TAIL)])

        plsc.subcore_barrier()

    run_relation(feat_user, src_f, dst_f, out_sf, out_cf)
    run_relation(feat_user, src_c, dst_c, out_sc, out_cc)
    run_relation(feat_item, src_b, dst_b, out_sb, out_cb)


_sc_agg = pl.kernel(
    _sc_body,
    out_type=[
        jax.ShapeDtypeStruct((NC, N_USER, D), jnp.float32),
        jax.ShapeDtypeStruct((NC, N_USER, CNT_W), jnp.float32),
        jax.ShapeDtypeStruct((NC, N_ITEM, D), jnp.float32),
        jax.ShapeDtypeStruct((NC, N_ITEM, CNT_W), jnp.float32),
        jax.ShapeDtypeStruct((NC, N_USER, D), jnp.float32),
        jax.ShapeDtypeStruct((NC, N_USER, CNT_W), jnp.float32),
    ],
    mesh=plsc.VectorSubcoreMesh(core_axis_name="c", subcore_axis_name="s"),
    compiler_params=pltpu.CompilerParams(use_tc_tiling_on_sc=False),
    scratch_types=[
        pltpu.VMEM_SHARED((N_USER, D), jnp.float32),
        pltpu.VMEM_SHARED((N_USER, CNT_W), jnp.float32),
        pltpu.VMEM((NCHUNK, CHUNK), jnp.int32),
        pltpu.VMEM((NCHUNK, CHUNK), jnp.int32),
        pltpu.VMEM((CHUNK, D), jnp.float32),
        pltpu.VMEM((CHUNK, CNT_W), jnp.float32),
        pltpu.SemaphoreType.DMA,
    ],
)

BR = 400
NBLK = N_USER // BR


def _tc_body(sf, cf, sb, cb, sc_r, cc, wf, bf, wb, bb, wc, bc, hu, hi):
    def rel(sref, cref, wref, bref):
        tot = sref[0] + sref[1]
        cnt = cref[0, :, 0:1] + cref[1, :, 0:1]
        mean = tot / jnp.maximum(cnt, 1.0)
        h = lax.dot_general(mean, wref[...], (((1,), (1,)), ((), ())),
                            preferred_element_type=jnp.float32,
                            precision=lax.Precision.HIGHEST) + bref[...]
        return jnp.where(cnt > 0, h, 0.0)

    hu[...] = rel(sf, cf, wf, bf) + rel(sb, cb, wb, bb)
    hi[...] = rel(sc_r, cc, wc, bc)


_sum_spec = pl.BlockSpec((NC, BR, D), lambda i: (0, i, 0))
_cnt_spec = pl.BlockSpec((NC, BR, CNT_W), lambda i: (0, i, 0))
_w_spec = pl.BlockSpec((D, D), lambda i: (0, 0))
_b_spec = pl.BlockSpec((1, D), lambda i: (0, 0))

_tc_finish = pl.pallas_call(
    _tc_body,
    grid=(NBLK,),
    in_specs=[_sum_spec, _cnt_spec, _sum_spec, _cnt_spec, _sum_spec, _cnt_spec,
              _w_spec, _b_spec, _w_spec, _b_spec, _w_spec, _b_spec],
    out_specs=[pl.BlockSpec((BR, D), lambda i: (i, 0)),
               pl.BlockSpec((BR, D), lambda i: (i, 0))],
    out_shape=[jax.ShapeDtypeStruct((N_USER, D), jnp.float32),
               jax.ShapeDtypeStruct((N_ITEM, D), jnp.float32)],
)


def kernel(feat_user, feat_item, W_follows, b_follows, W_clicks, b_clicks,
           W_bought, b_bought, edge_index_follows, edge_index_clicks,
           edge_index_bought):
    def split_edges(ei):
        return (ei[0].reshape(NW, NCHUNK, CHUNK), ei[1].reshape(NW, NCHUNK, CHUNK))

    sf_, df_ = split_edges(edge_index_follows)
    sc_, dc_ = split_edges(edge_index_clicks)
    sb_, db_ = split_edges(edge_index_bought)

    z_s = jnp.zeros((RPT, D), jnp.float32)
    z_c = jnp.zeros((RPT, CNT_W), jnp.float32)
    sum_f, cnt_f, sum_c, cnt_c, sum_b, cnt_b = _sc_agg(
        feat_user, feat_item, z_s, z_c, sf_, df_, sc_, dc_, sb_, db_)

    h_user, h_item = _tc_finish(
        sum_f, cnt_f, sum_b, cnt_b, sum_c, cnt_c,
        W_follows, b_follows.reshape(1, D),
        W_bought, b_bought.reshape(1, D),
        W_clicks, b_clicks.reshape(1, D))
    return (h_user, h_item)
